# Initial kernel scaffold; baseline (speedup 1.0000x reference)
import jax
import jax.numpy as jnp
from jax.experimental import pallas as pl


def kernel(x, encoder, encoder_bias, decoder, decoder_bias):
    raise NotImplementedError("write your pallas kernel here")


# speedup vs baseline: 11.6797x; 11.6797x over previous
import jax
import jax.numpy as jnp
from jax.experimental import pallas as pl
from jax.experimental.pallas import tpu as pltpu

_TOPK = 64
_I32_MAX = 0x7FFFFFFF


def _sortable_key(pre):
    bi = jax.lax.bitcast_convert_type(pre, jnp.int32)
    return jnp.where(bi >= 0, bi, bi ^ jnp.int32(_I32_MAX))


def _mm1_kernel(x_ref, e_ref, b_ref, o_ref):
    o_ref[...] = (
        jnp.dot(x_ref[...], e_ref[...], preferred_element_type=jnp.float32)
        + b_ref[...]
    )


def _tkey_kernel(pre_ref, t_ref):
    key = _sortable_key(pre_ref[...])
    rows = key.shape[0]
    lo0 = jnp.full((rows, 1), -1, jnp.int32)
    hi0 = jnp.full((rows, 1), _I32_MAX, jnp.int32)

    def body(_, lohi):
        lo, hi = lohi
        mid = (lo >> 1) + (hi >> 1) + (lo & hi & 1)
        cnt = jnp.sum((key > mid).astype(jnp.int32), axis=1, keepdims=True)
        le = cnt <= (_TOPK - 1)
        return jnp.where(le, lo, mid + 1), jnp.where(le, mid, hi)

    _, hi = jax.lax.fori_loop(0, 32, body, (lo0, hi0))
    t_ref[...] = hi


def _decode_kernel(pre_ref, t_ref, d_ref, b_ref, o_ref):
    kstep = pl.program_id(1)
    pre = pre_ref[...]
    keep = (_sortable_key(pre) >= t_ref[...]) & (pre > 0)
    acts = jnp.where(keep, pre, 0.0)
    part = jnp.dot(acts, d_ref[...], preferred_element_type=jnp.float32)

    @pl.when(kstep == 0)
    def _():
        o_ref[...] = jnp.broadcast_to(b_ref[...], o_ref.shape)

    o_ref[...] += part


def kernel(x, encoder, encoder_bias, decoder, decoder_bias):
    m, d_model = x.shape
    d_hidden = encoder.shape[1]

    bm1, bn1 = 512, 2048
    pre = pl.pallas_call(
        _mm1_kernel,
        grid=(d_hidden // bn1, m // bm1),
        in_specs=[
            pl.BlockSpec((bm1, d_model), lambda j, i: (i, 0)),
            pl.BlockSpec((d_model, bn1), lambda j, i: (0, j)),
            pl.BlockSpec((1, bn1), lambda j, i: (0, j)),
        ],
        out_specs=pl.BlockSpec((bm1, bn1), lambda j, i: (i, j)),
        out_shape=jax.ShapeDtypeStruct((m, d_hidden), jnp.float32),
        compiler_params=pltpu.CompilerParams(
            dimension_semantics=("parallel", "parallel")
        ),
    )(x, encoder, encoder_bias.reshape(1, -1))

    bmt = 256
    tkey = pl.pallas_call(
        _tkey_kernel,
        grid=(m // bmt,),
        in_specs=[pl.BlockSpec((bmt, d_hidden), lambda i: (i, 0))],
        out_specs=pl.BlockSpec((bmt, 1), lambda i: (i, 0)),
        out_shape=jax.ShapeDtypeStruct((m, 1), jnp.int32),
        compiler_params=pltpu.CompilerParams(
            dimension_semantics=("parallel",)
        ),
    )(pre)

    bm2, bk2 = 1024, 1024
    recon = pl.pallas_call(
        _decode_kernel,
        grid=(m // bm2, d_hidden // bk2),
        in_specs=[
            pl.BlockSpec((bm2, bk2), lambda i, k: (i, k)),
            pl.BlockSpec((bm2, 1), lambda i, k: (i, 0)),
            pl.BlockSpec((bk2, d_model), lambda i, k: (k, 0)),
            pl.BlockSpec((1, d_model), lambda i, k: (0, 0)),
        ],
        out_specs=pl.BlockSpec((bm2, d_model), lambda i, k: (i, 0)),
        out_shape=jax.ShapeDtypeStruct((m, d_model), jnp.float32),
        compiler_params=pltpu.CompilerParams(
            dimension_semantics=("parallel", "arbitrary")
        ),
    )(pre, tkey, decoder, decoder_bias.reshape(1, -1))

    return recon
